# baseline (speedup 1.0000x reference)
import functools

import jax
import jax.numpy as jnp
from jax import lax
from jax.experimental import pallas as pl
from jax.experimental.pallas import tpu as pltpu
from jax.experimental.pallas import tpu_sc as plsc

_NC, _NS = 2, 16
_NW = _NC * _NS
_BLK = 128
_T = 512


def _sc_mesh():
    return plsc.VectorSubcoreMesh(core_axis_name="c", subcore_axis_name="s",
                                  num_cores=_NC, num_subcores=_NS)


def _make_scatter(n, in_dim, npad, nfull, tail, nsteps):
    scratch = [
        pltpu.VMEM((_BLK,), jnp.int32),
        pltpu.VMEM((_BLK, in_dim), jnp.float32),
        pltpu.VMEM((max(tail, 8),), jnp.int32),
        pltpu.VMEM((max(tail, 8), in_dim), jnp.float32),
        pltpu.SemaphoreType.DMA,
    ]

    @functools.partial(
        pl.kernel,
        out_type=jax.ShapeDtypeStruct((npad, in_dim), jnp.float32),
        mesh=_sc_mesh(),
        scratch_types=scratch,
    )
    def scatter(x_hbm, pos_hbm, xs_hbm, idx_v, rows_v, idxt_v, rowst_v, sem):
        wid = lax.axis_index("s") * _NC + lax.axis_index("c")

        def step(j, carry):
            b = wid + _NW * j

            @pl.when(b < nfull)
            def _():
                off = b * _BLK
                pltpu.sync_copy(pos_hbm.at[pl.ds(off, _BLK)], idx_v)
                pltpu.sync_copy(x_hbm.at[pl.ds(off, _BLK), :], rows_v)
                pltpu.async_copy(rows_v, xs_hbm.at[idx_v], sem).wait()

            if tail:
                @pl.when(b == nfull)
                def _():
                    off = nfull * _BLK
                    pltpu.sync_copy(pos_hbm.at[pl.ds(off, tail)],
                                    idxt_v.at[pl.ds(0, tail)])
                    pltpu.sync_copy(x_hbm.at[pl.ds(off, tail), :],
                                    rowst_v.at[pl.ds(0, tail), :])
                    pltpu.async_copy(rowst_v.at[pl.ds(0, tail), :],
                                     xs_hbm.at[idxt_v.at[pl.ds(0, tail)]],
                                     sem).wait()

            return carry

        lax.fori_loop(0, nsteps, step, 0)

    return scatter


def _make_gather(n, npad, nfull, tail, nsteps):
    scratch = [
        pltpu.VMEM((_BLK,), jnp.int32),
        pltpu.VMEM((_BLK,), jnp.float32),
        pltpu.VMEM((max(tail, 8),), jnp.int32),
        pltpu.VMEM((max(tail, 8),), jnp.float32),
        pltpu.SemaphoreType.DMA,
    ]

    @functools.partial(
        pl.kernel,
        out_type=jax.ShapeDtypeStruct((n,), jnp.float32),
        mesh=_sc_mesh(),
        scratch_types=scratch,
    )
    def gather(ys_hbm, pos_hbm, out_hbm, idx_v, y_v, idxt_v, yt_v, sem):
        wid = lax.axis_index("s") * _NC + lax.axis_index("c")

        def step(j, carry):
            b = wid + _NW * j

            @pl.when(b < nfull)
            def _():
                off = b * _BLK
                pltpu.sync_copy(pos_hbm.at[pl.ds(off, _BLK)], idx_v)
                pltpu.async_copy(ys_hbm.at[idx_v], y_v, sem).wait()
                pltpu.sync_copy(y_v, out_hbm.at[pl.ds(off, _BLK)])

            if tail:
                @pl.when(b == nfull)
                def _():
                    off = nfull * _BLK
                    pltpu.sync_copy(pos_hbm.at[pl.ds(off, tail)],
                                    idxt_v.at[pl.ds(0, tail)])
                    pltpu.async_copy(ys_hbm.at[idxt_v.at[pl.ds(0, tail)]],
                                     yt_v.at[pl.ds(0, tail)], sem).wait()
                    pltpu.sync_copy(yt_v.at[pl.ds(0, tail)],
                                    out_hbm.at[pl.ds(off, tail)])

            return carry

        lax.fori_loop(0, nsteps, step, 0)

    return gather


def _mlp_body(te_ref, xs_ref, w1_ref, b1_ref, w2_ref, b2_ref, o_ref):
    xb = xs_ref[...].astype(jnp.bfloat16)
    hT = lax.dot_general(w1_ref[0], xb, (((1,), (1,)), ((), ())),
                         preferred_element_type=jnp.float32)
    hT = jnp.maximum(hT + b1_ref[0, 0][:, None], 0.0).astype(jnp.bfloat16)
    oT = lax.dot_general(w2_ref[0, 0][None, :], hT, (((1,), (0,)), ((), ())),
                         preferred_element_type=jnp.float32)
    o_ref[0] = oT + b2_ref[0, 0, 0]


def kernel(x, atomic_nums, Wc1, bc1, Wc2, bc2, Wh1, bh1, Wh2, bh2,
           Wo1, bo1, Wo2, bo2):
    n, in_dim = x.shape
    hid = Wc1.shape[0]
    ntiles = (n + _T - 1) // _T + 2
    npad = ntiles * _T
    nfull = n // _BLK
    tail = n - nfull * _BLK
    nblocks = nfull + (1 if tail else 0)
    nsteps = (nblocks + _NW - 1) // _NW

    an = atomic_nums.astype(jnp.int32)
    is0 = an == 6
    is1 = an == 1
    c0 = jnp.sum(is0.astype(jnp.int32))
    c1 = jnp.sum(is1.astype(jnp.int32))
    s1 = ((c0 + _T - 1) // _T) * _T
    s2 = s1 + ((c1 + _T - 1) // _T) * _T
    cum0 = jnp.cumsum(is0.astype(jnp.int32))
    cum1 = jnp.cumsum(is1.astype(jnp.int32))
    cum2 = jnp.cumsum(jnp.logical_and(~is0, ~is1).astype(jnp.int32))
    pos = jnp.where(is0, cum0 - 1,
                    jnp.where(is1, s1 + cum1 - 1, s2 + cum2 - 1)).astype(jnp.int32)
    tstart = jnp.arange(ntiles, dtype=jnp.int32) * _T
    te = ((tstart >= s1).astype(jnp.int32) + (tstart >= s2).astype(jnp.int32))

    xs = _make_scatter(n, in_dim, npad, nfull, tail, nsteps)(x, pos)

    w1s = jnp.stack([Wc1, Wh1, Wo1]).astype(jnp.bfloat16)
    b1s = jnp.stack([bc1, bh1, bo1]).reshape(3, 1, hid)
    w2s = jnp.stack([Wc2[0], Wh2[0], Wo2[0]]).astype(jnp.bfloat16).reshape(3, 1, hid)
    b2s = jnp.stack([bc2, bh2, bo2]).reshape(3, 1, 1)

    grid_spec = pltpu.PrefetchScalarGridSpec(
        num_scalar_prefetch=1,
        grid=(ntiles,),
        in_specs=[
            pl.BlockSpec((_T, in_dim), lambda i, te_r: (i, 0)),
            pl.BlockSpec((1, hid, in_dim), lambda i, te_r: (te_r[i], 0, 0)),
            pl.BlockSpec((1, 1, hid), lambda i, te_r: (te_r[i], 0, 0)),
            pl.BlockSpec((1, 1, hid), lambda i, te_r: (te_r[i], 0, 0)),
            pl.BlockSpec((1, 1, 1), lambda i, te_r: (te_r[i], 0, 0)),
        ],
        out_specs=pl.BlockSpec((1, 1, _T), lambda i, te_r: (i, 0, 0)),
    )
    ys = pl.pallas_call(
        _mlp_body,
        grid_spec=grid_spec,
        out_shape=jax.ShapeDtypeStruct((ntiles, 1, _T), jnp.float32),
    )(te, xs, w1s, b1s, w2s, b2s)
    ys = ys.reshape(npad)

    out = _make_gather(n, npad, nfull, tail, nsteps)(ys, pos)
    return out.reshape(n, 1)


# speedup vs baseline: 1.5340x; 1.5340x over previous
import functools

import jax
import jax.numpy as jnp
from jax import lax
from jax.experimental import pallas as pl
from jax.experimental.pallas import tpu as pltpu
from jax.experimental.pallas import tpu_sc as plsc

_NC, _NS = 2, 16
_NW = _NC * _NS
_BLK = 128
_SB = 8
_T = 1024


def _sc_mesh():
    return plsc.VectorSubcoreMesh(core_axis_name="c", subcore_axis_name="s",
                                  num_cores=_NC, num_subcores=_NS)


def _make_scatter(n, in_dim, npad, nfull, tail, nsteps):
    scratch = [
        pltpu.VMEM((2, _BLK), jnp.int32),
        pltpu.VMEM((2, _BLK, in_dim), jnp.float32),
        pltpu.VMEM((max(tail, 8),), jnp.int32),
        pltpu.VMEM((max(tail, 8), in_dim), jnp.float32),
        pltpu.SemaphoreType.DMA,
        pltpu.SemaphoreType.DMA,
        pltpu.SemaphoreType.DMA,
        pltpu.SemaphoreType.DMA,
        pltpu.SemaphoreType.DMA,
    ]

    @functools.partial(
        pl.kernel,
        out_type=jax.ShapeDtypeStruct((npad, in_dim), jnp.float32),
        mesh=_sc_mesh(),
        scratch_types=scratch,
    )
    def scatter(x_hbm, pos_hbm, xs_hbm, idx_v, rows_v, idxt_v, rowst_v,
                ls0, ls1, ss0, ss1, tsem):
        wid = lax.axis_index("s") * _NC + lax.axis_index("c")
        lsem = (ls0, ls1)
        ssem = (ss0, ss1)

        def load_descs(j, p):
            off = (wid + _NW * j) * _BLK
            di = pltpu.make_async_copy(pos_hbm.at[pl.ds(off, _BLK)],
                                       idx_v.at[p], lsem[p])
            dr = pltpu.make_async_copy(x_hbm.at[pl.ds(off, _BLK), :],
                                       rows_v.at[p], lsem[p])
            return di, dr

        def scat_desc(p):
            return pltpu.make_async_copy(rows_v.at[p], xs_hbm.at[idx_v.at[p]],
                                         ssem[p])

        @pl.when(wid < nfull)
        def _():
            di, dr = load_descs(0, 0)
            di.start()
            dr.start()

        def half_step(j, p):
            b = wid + _NW * j

            @pl.when(jnp.logical_and(j >= 1, b - _NW < nfull))
            def _():
                scat_desc(1 - p).wait()

            @pl.when(b + _NW < nfull)
            def _():
                di, dr = load_descs(j + 1, 1 - p)
                di.start()
                dr.start()

            @pl.when(b < nfull)
            def _():
                di, dr = load_descs(j, p)
                di.wait()
                dr.wait()
                scat_desc(p).start()

        def step(jp, carry):
            half_step(2 * jp, 0)
            half_step(2 * jp + 1, 1)
            return carry

        lax.fori_loop(0, (nsteps + 2) // 2, step, 0)

        if tail:
            @pl.when(wid == (nfull % _NW))
            def _():
                off = nfull * _BLK
                pltpu.sync_copy(pos_hbm.at[pl.ds(off, tail)],
                                idxt_v.at[pl.ds(0, tail)])
                pltpu.sync_copy(x_hbm.at[pl.ds(off, tail), :],
                                rowst_v.at[pl.ds(0, tail), :])
                pltpu.async_copy(rowst_v.at[pl.ds(0, tail), :],
                                 xs_hbm.at[idxt_v.at[pl.ds(0, tail)]],
                                 tsem).wait()

    return scatter


def _make_gather(nsb, npad):
    scratch = [
        pltpu.VMEM((_SB, _BLK), jnp.int32),
        pltpu.VMEM((_SB, _BLK), jnp.float32),
        pltpu.SemaphoreType.DMA,
    ]
    ksteps = (nsb + _NW - 1) // _NW

    @functools.partial(
        pl.kernel,
        out_type=jax.ShapeDtypeStruct((nsb, _SB, _BLK), jnp.float32),
        mesh=_sc_mesh(),
        scratch_types=scratch,
    )
    def gather(ys_hbm, pos3_hbm, out_hbm, idx_v, y_v, sem):
        wid = lax.axis_index("s") * _NC + lax.axis_index("c")

        def step(k, carry):
            s = wid + _NW * k

            @pl.when(s < nsb)
            def _():
                pltpu.sync_copy(pos3_hbm.at[s], idx_v)
                for kk in range(_SB):
                    pltpu.make_async_copy(ys_hbm.at[idx_v.at[kk]],
                                          y_v.at[kk], sem).start()
                for kk in range(_SB):
                    pltpu.make_async_copy(ys_hbm.at[idx_v.at[kk]],
                                          y_v.at[kk], sem).wait()
                pltpu.sync_copy(y_v, out_hbm.at[s])

            return carry

        lax.fori_loop(0, ksteps, step, 0)

    return gather


def _mlp_body(te_ref, xs_ref, w1_ref, b1_ref, w2_ref, b2_ref, o_ref):
    xb = xs_ref[...].astype(jnp.bfloat16)
    hT = lax.dot_general(w1_ref[0], xb, (((1,), (1,)), ((), ())),
                         preferred_element_type=jnp.float32)
    hT = jnp.maximum(hT + b1_ref[0, 0][:, None], 0.0).astype(jnp.bfloat16)
    oT = lax.dot_general(w2_ref[0, 0][None, :], hT, (((1,), (0,)), ((), ())),
                         preferred_element_type=jnp.float32)
    o_ref[0] = oT + b2_ref[0, 0, 0]


def kernel(x, atomic_nums, Wc1, bc1, Wc2, bc2, Wh1, bh1, Wh2, bh2,
           Wo1, bo1, Wo2, bo2):
    n, in_dim = x.shape
    hid = Wc1.shape[0]
    ntiles = (n + _T - 1) // _T + 2
    npad = ntiles * _T
    nfull = n // _BLK
    tail = n - nfull * _BLK
    nblocks = nfull + (1 if tail else 0)
    nsteps = (nblocks + _NW - 1) // _NW
    sbrows = _SB * _BLK
    nsb = (n + sbrows - 1) // sbrows
    ngpad = nsb * sbrows

    an = atomic_nums.astype(jnp.int32)
    is0 = an == 6
    is1 = an == 1
    c0 = jnp.sum(is0.astype(jnp.int32))
    c1 = jnp.sum(is1.astype(jnp.int32))
    s1 = ((c0 + _T - 1) // _T) * _T
    s2 = s1 + ((c1 + _T - 1) // _T) * _T
    cum0 = jnp.cumsum(is0.astype(jnp.int32))
    cum1 = jnp.cumsum(is1.astype(jnp.int32))
    iota1 = jnp.arange(1, n + 1, dtype=jnp.int32)
    pos = jnp.where(is0, cum0 - 1,
                    jnp.where(is1, s1 + cum1 - 1,
                              s2 + (iota1 - cum0 - cum1) - 1)).astype(jnp.int32)
    tstart = jnp.arange(ntiles, dtype=jnp.int32) * _T
    te = ((tstart >= s1).astype(jnp.int32) + (tstart >= s2).astype(jnp.int32))
    pos3 = jnp.concatenate(
        [pos, jnp.full((ngpad - n,), npad - 1, jnp.int32)]).reshape(nsb, _SB, _BLK)

    xs = _make_scatter(n, in_dim, npad, nfull, tail, nsteps)(x, pos)

    w1s = jnp.stack([Wc1, Wh1, Wo1]).astype(jnp.bfloat16)
    b1s = jnp.stack([bc1, bh1, bo1]).reshape(3, 1, hid)
    w2s = jnp.stack([Wc2[0], Wh2[0], Wo2[0]]).astype(jnp.bfloat16).reshape(3, 1, hid)
    b2s = jnp.stack([bc2, bh2, bo2]).reshape(3, 1, 1)

    grid_spec = pltpu.PrefetchScalarGridSpec(
        num_scalar_prefetch=1,
        grid=(ntiles,),
        in_specs=[
            pl.BlockSpec((_T, in_dim), lambda i, te_r: (i, 0)),
            pl.BlockSpec((1, hid, in_dim), lambda i, te_r: (te_r[i], 0, 0)),
            pl.BlockSpec((1, 1, hid), lambda i, te_r: (te_r[i], 0, 0)),
            pl.BlockSpec((1, 1, hid), lambda i, te_r: (te_r[i], 0, 0)),
            pl.BlockSpec((1, 1, 1), lambda i, te_r: (te_r[i], 0, 0)),
        ],
        out_specs=pl.BlockSpec((1, 1, _T), lambda i, te_r: (i, 0, 0)),
    )
    ys = pl.pallas_call(
        _mlp_body,
        grid_spec=grid_spec,
        out_shape=jax.ShapeDtypeStruct((ntiles, 1, _T), jnp.float32),
    )(te, xs, w1s, b1s, w2s, b2s)
    ys = ys.reshape(npad)

    out3 = _make_gather(nsb, npad)(ys, pos3)
    return out3.reshape(ngpad)[:n].reshape(n, 1)


# speedup vs baseline: 1.8531x; 1.2080x over previous
import functools

import jax
import jax.numpy as jnp
from jax import lax
from jax.experimental import pallas as pl
from jax.experimental.pallas import tpu as pltpu
from jax.experimental.pallas import tpu_sc as plsc

_NC, _NS = 2, 16
_NW = _NC * _NS
_BLK = 128
_SB = 8
_T = 4000


def _sc_mesh():
    return plsc.VectorSubcoreMesh(core_axis_name="c", subcore_axis_name="s",
                                  num_cores=_NC, num_subcores=_NS)


def _make_scatter(n, in_dim, npad, nfull, tail, nsteps):
    scratch = [
        pltpu.VMEM((2, _BLK), jnp.int32),
        pltpu.VMEM((2, _BLK, in_dim), jnp.float32),
        pltpu.VMEM((max(tail, 8),), jnp.int32),
        pltpu.VMEM((max(tail, 8), in_dim), jnp.float32),
        pltpu.SemaphoreType.DMA,
        pltpu.SemaphoreType.DMA,
        pltpu.SemaphoreType.DMA,
        pltpu.SemaphoreType.DMA,
        pltpu.SemaphoreType.DMA,
    ]

    @functools.partial(
        pl.kernel,
        out_type=jax.ShapeDtypeStruct((npad, in_dim), jnp.float32),
        mesh=_sc_mesh(),
        scratch_types=scratch,
    )
    def scatter(x_hbm, pos_hbm, xs_hbm, idx_v, rows_v, idxt_v, rowst_v,
                ls0, ls1, ss0, ss1, tsem):
        wid = lax.axis_index("s") * _NC + lax.axis_index("c")
        lsem = (ls0, ls1)
        ssem = (ss0, ss1)

        def load_descs(j, p):
            off = (wid + _NW * j) * _BLK
            di = pltpu.make_async_copy(pos_hbm.at[pl.ds(off, _BLK)],
                                       idx_v.at[p], lsem[p])
            dr = pltpu.make_async_copy(x_hbm.at[pl.ds(off, _BLK), :],
                                       rows_v.at[p], lsem[p])
            return di, dr

        def scat_desc(p):
            return pltpu.make_async_copy(rows_v.at[p], xs_hbm.at[idx_v.at[p]],
                                         ssem[p])

        @pl.when(wid < nfull)
        def _():
            di, dr = load_descs(0, 0)
            di.start()
            dr.start()

        def half_step(j, p):
            b = wid + _NW * j

            @pl.when(jnp.logical_and(j >= 1, b - _NW < nfull))
            def _():
                scat_desc(1 - p).wait()

            @pl.when(b + _NW < nfull)
            def _():
                di, dr = load_descs(j + 1, 1 - p)
                di.start()
                dr.start()

            @pl.when(b < nfull)
            def _():
                di, dr = load_descs(j, p)
                di.wait()
                dr.wait()
                scat_desc(p).start()

        def step(jp, carry):
            half_step(2 * jp, 0)
            half_step(2 * jp + 1, 1)
            return carry

        lax.fori_loop(0, (nsteps + 2) // 2, step, 0)

        if tail:
            @pl.when(wid == (nfull % _NW))
            def _():
                off = nfull * _BLK
                pltpu.sync_copy(pos_hbm.at[pl.ds(off, tail)],
                                idxt_v.at[pl.ds(0, tail)])
                pltpu.sync_copy(x_hbm.at[pl.ds(off, tail), :],
                                rowst_v.at[pl.ds(0, tail), :])
                pltpu.async_copy(rowst_v.at[pl.ds(0, tail), :],
                                 xs_hbm.at[idxt_v.at[pl.ds(0, tail)]],
                                 tsem).wait()

    return scatter


def _make_gather(nsb, npad):
    scratch = [
        pltpu.VMEM((_SB, _BLK), jnp.int32),
        pltpu.VMEM((_SB, _BLK), jnp.float32),
        pltpu.SemaphoreType.DMA,
    ]
    ksteps = (nsb + _NW - 1) // _NW

    @functools.partial(
        pl.kernel,
        out_type=jax.ShapeDtypeStruct((nsb, _SB, _BLK), jnp.float32),
        mesh=_sc_mesh(),
        scratch_types=scratch,
    )
    def gather(ys_hbm, pos3_hbm, out_hbm, idx_v, y_v, sem):
        wid = lax.axis_index("s") * _NC + lax.axis_index("c")

        def step(k, carry):
            s = wid + _NW * k

            @pl.when(s < nsb)
            def _():
                pltpu.sync_copy(pos3_hbm.at[s], idx_v)
                for kk in range(_SB):
                    pltpu.make_async_copy(ys_hbm.at[idx_v.at[kk]],
                                          y_v.at[kk], sem).start()
                for kk in range(_SB):
                    pltpu.make_async_copy(ys_hbm.at[idx_v.at[kk]],
                                          y_v.at[kk], sem).wait()
                pltpu.sync_copy(y_v, out_hbm.at[s])

            return carry

        lax.fori_loop(0, ksteps, step, 0)

    return gather


def _mlp_body(te_ref, xs_ref, w1_ref, b1_ref, w2_ref, b2_ref, o_ref):
    xb = xs_ref[...].astype(jnp.bfloat16)
    hT = lax.dot_general(w1_ref[0], xb, (((1,), (1,)), ((), ())),
                         preferred_element_type=jnp.float32)
    hT = jnp.maximum(hT + b1_ref[0, 0][:, None], 0.0).astype(jnp.bfloat16)
    oT = lax.dot_general(w2_ref[0, 0][None, :], hT, (((1,), (0,)), ((), ())),
                         preferred_element_type=jnp.float32)
    o_ref[0] = oT + b2_ref[0, 0, 0]


def kernel(x, atomic_nums, Wc1, bc1, Wc2, bc2, Wh1, bh1, Wh2, bh2,
           Wo1, bo1, Wo2, bo2):
    n, in_dim = x.shape
    hid = Wc1.shape[0]
    ntiles = (n + _T - 1) // _T + 2
    npad = ntiles * _T
    nfull = n // _BLK
    tail = n - nfull * _BLK
    nblocks = nfull + (1 if tail else 0)
    nsteps = (nblocks + _NW - 1) // _NW
    sbrows = _SB * _BLK
    nsb = (n + sbrows - 1) // sbrows
    ngpad = nsb * sbrows

    an = atomic_nums.astype(jnp.int32)
    is0 = an == 6
    is1 = an == 1
    c0 = jnp.sum(is0.astype(jnp.int32))
    c1 = jnp.sum(is1.astype(jnp.int32))
    s1 = ((c0 + _T - 1) // _T) * _T
    s2 = s1 + ((c1 + _T - 1) // _T) * _T
    cum0 = jnp.cumsum(is0.astype(jnp.int32))
    cum1 = jnp.cumsum(is1.astype(jnp.int32))
    iota1 = jnp.arange(1, n + 1, dtype=jnp.int32)
    pos = jnp.where(is0, cum0 - 1,
                    jnp.where(is1, s1 + cum1 - 1,
                              s2 + (iota1 - cum0 - cum1) - 1)).astype(jnp.int32)
    tstart = jnp.arange(ntiles, dtype=jnp.int32) * _T
    te = ((tstart >= s1).astype(jnp.int32) + (tstart >= s2).astype(jnp.int32))
    pos3 = jnp.concatenate(
        [pos, jnp.full((ngpad - n,), npad - 1, jnp.int32)]).reshape(nsb, _SB, _BLK)

    xs = _make_scatter(n, in_dim, npad, nfull, tail, nsteps)(x, pos)

    w1s = jnp.stack([Wc1, Wh1, Wo1]).astype(jnp.bfloat16)
    b1s = jnp.stack([bc1, bh1, bo1]).reshape(3, 1, hid)
    w2s = jnp.stack([Wc2[0], Wh2[0], Wo2[0]]).astype(jnp.bfloat16).reshape(3, 1, hid)
    b2s = jnp.stack([bc2, bh2, bo2]).reshape(3, 1, 1)

    grid_spec = pltpu.PrefetchScalarGridSpec(
        num_scalar_prefetch=1,
        grid=(ntiles,),
        in_specs=[
            pl.BlockSpec((_T, in_dim), lambda i, te_r: (i, 0)),
            pl.BlockSpec((1, hid, in_dim), lambda i, te_r: (te_r[i], 0, 0)),
            pl.BlockSpec((1, 1, hid), lambda i, te_r: (te_r[i], 0, 0)),
            pl.BlockSpec((1, 1, hid), lambda i, te_r: (te_r[i], 0, 0)),
            pl.BlockSpec((1, 1, 1), lambda i, te_r: (te_r[i], 0, 0)),
        ],
        out_specs=pl.BlockSpec((1, 1, _T), lambda i, te_r: (i, 0, 0)),
    )
    ys = pl.pallas_call(
        _mlp_body,
        grid_spec=grid_spec,
        compiler_params=pltpu.CompilerParams(
            dimension_semantics=("arbitrary",)),
        out_shape=jax.ShapeDtypeStruct((ntiles, 1, _T), jnp.float32),
    )(te, xs, w1s, b1s, w2s, b2s)
    ys = ys.reshape(npad)

    out3 = _make_gather(nsb, npad)(ys, pos3)
    return out3.reshape(ngpad)[:n].reshape(n, 1)
